# baseline (speedup 1.0000x reference)
import functools

import jax
import jax.numpy as jnp
from jax import lax
from jax.experimental import pallas as pl
from jax.experimental.pallas import tpu as pltpu
from jax.experimental.pallas import tpu_sc as plsc

B = 16384
D = 128
CHUNK = 128
NW = 32
BPW = B // NW
NCHUNK = BPW // CHUNK
T = 2 * NCHUNK
NBUF = 7

R_PAD = 512
BB = 512


def _gather_entities(qe, oe, ent_table):
    mesh = plsc.VectorSubcoreMesh(core_axis_name="c", subcore_axis_name="s")
    out_type = (
        jax.ShapeDtypeStruct((B, D), jnp.float32),
        jax.ShapeDtypeStruct((B, D), jnp.float32),
    )
    scratch = (
        [pltpu.VMEM((NCHUNK, CHUNK), jnp.int32)] * 2
        + [pltpu.VMEM((CHUNK, D), jnp.float32)] * NBUF
        + [pltpu.SemaphoreType.DMA] * (1 + 2 * NBUF)
    )

    @functools.partial(pl.kernel, mesh=mesh, out_type=out_type,
                       scratch_types=scratch)
    def k(qe_hbm, oe_hbm, ent_hbm, out_qe, out_oe, *scr):
        qe_v, oe_v = scr[0:2]
        bufs = scr[2:2 + NBUF]
        isem = scr[2 + NBUF]
        gsem = scr[3 + NBUF:3 + 2 * NBUF]
        ssem = scr[3 + 2 * NBUF:3 + 3 * NBUF]

        wid = lax.axis_index("s") * 2 + lax.axis_index("c")
        row0 = wid * NCHUNK
        c1 = pltpu.async_copy(qe_hbm.at[pl.ds(row0, NCHUNK)], qe_v, isem)
        c2 = pltpu.async_copy(oe_hbm.at[pl.ds(row0, NCHUNK)], oe_v, isem)
        c1.wait(); c2.wait()

        tasks = []
        for iv, out in ((qe_v, out_qe), (oe_v, out_oe)):
            for j in range(NCHUNK):
                tasks.append((iv.at[j], out, (row0 + j) * CHUNK))

        gcp = [None] * T
        scp = [None] * T
        for t in range(NBUF):
            iv_row, _, _ = tasks[t]
            gcp[t] = pltpu.async_copy(ent_hbm.at[iv_row], bufs[t], gsem[t])
        for t in range(T):
            b = t % NBUF
            _, out, off = tasks[t]
            gcp[t].wait()
            scp[t] = pltpu.async_copy(bufs[b], out.at[pl.ds(off, CHUNK)],
                                      ssem[b])
            nt = t + NBUF
            if nt < T:
                scp[t].wait()
                iv_row, _, _ = tasks[nt]
                gcp[nt] = pltpu.async_copy(ent_hbm.at[iv_row], bufs[b],
                                           gsem[b])
        for t in range(max(0, T - NBUF), T):
            scp[t].wait()

    return k(qe, oe, ent_table)


def _rel_body(idx_ref, hi_ref, lo_ref, out_ref):
    idx_col = idx_ref[:]
    r = lax.broadcasted_iota(jnp.int32, (BB, R_PAD), 1)
    oh = (r == idx_col).astype(jnp.bfloat16)
    out_ref[:] = (
        jnp.dot(oh, hi_ref[:], preferred_element_type=jnp.float32)
        + jnp.dot(oh, lo_ref[:], preferred_element_type=jnp.float32)
    )


def _rel_lookup(qr, rel_table):
    hi = rel_table.astype(jnp.bfloat16)
    lo = (rel_table - hi.astype(jnp.float32)).astype(jnp.bfloat16)
    pad = ((0, R_PAD - rel_table.shape[0]), (0, 0))
    hi = jnp.pad(hi, pad)
    lo = jnp.pad(lo, pad)
    idx = qr.astype(jnp.int32).reshape(B, 1)
    return pl.pallas_call(
        _rel_body,
        grid=(B // BB,),
        in_specs=[
            pl.BlockSpec((BB, 1), lambda i: (i, 0)),
            pl.BlockSpec((R_PAD, D), lambda i: (0, 0)),
            pl.BlockSpec((R_PAD, D), lambda i: (0, 0)),
        ],
        out_specs=pl.BlockSpec((BB, D), lambda i: (i, 0)),
        out_shape=jax.ShapeDtypeStruct((B, D), jnp.float32),
    )(idx, hi, lo)


def kernel(query_entities, query_relations, obj_entities, ent_table, rel_table):
    qe = query_entities.astype(jnp.int32).reshape(B // CHUNK, CHUNK)
    oe = obj_entities.astype(jnp.int32).reshape(B // CHUNK, CHUNK)
    out_qe, out_oe = _gather_entities(qe, oe, ent_table)
    out_qr = _rel_lookup(query_relations, rel_table)
    return (out_qe, out_qr, out_oe)


# speedup vs baseline: 1.5242x; 1.5242x over previous
import functools

import jax
import jax.numpy as jnp
from jax import lax
from jax.experimental import pallas as pl
from jax.experimental.pallas import tpu as pltpu
from jax.experimental.pallas import tpu_sc as plsc

B = 16384
D = 128
N_REL_ROWS = 500
CHUNK = 128
NW = 32
BPW = B // NW
NCHUNK = BPW // CHUNK
TE = 2 * NCHUNK
T = 3 * NCHUNK
NBUF = 6


def _gather3(qe, qr, oe, ent_table, rel_table):
    mesh = plsc.VectorSubcoreMesh(core_axis_name="c", subcore_axis_name="s")
    out_type = (
        jax.ShapeDtypeStruct((B, D), jnp.float32),
        jax.ShapeDtypeStruct((B, D), jnp.float32),
        jax.ShapeDtypeStruct((B, D), jnp.float32),
    )
    scratch = (
        [pltpu.VMEM((NCHUNK, CHUNK), jnp.int32)] * 3
        + [pltpu.VMEM((CHUNK, D), jnp.float32)] * NBUF
        + [pltpu.VMEM_SHARED((N_REL_ROWS, D), jnp.float32)]
        + [pltpu.SemaphoreType.DMA] * (2 + 2 * NBUF)
    )

    @functools.partial(pl.kernel, mesh=mesh, out_type=out_type,
                       scratch_types=scratch)
    def k(qe_hbm, qr_hbm, oe_hbm, ent_hbm, rel_hbm,
          out_qe, out_qr, out_oe, *scr):
        qe_v, qr_v, oe_v = scr[0:3]
        bufs = scr[3:3 + NBUF]
        rel_sp = scr[3 + NBUF]
        isem = scr[4 + NBUF]
        rsem = scr[5 + NBUF]
        gsem = scr[6 + NBUF:6 + 2 * NBUF]
        ssem = scr[6 + 2 * NBUF:6 + 3 * NBUF]

        sid = lax.axis_index("s")
        wid = sid * 2 + lax.axis_index("c")
        row0 = wid * NCHUNK

        @pl.when(sid == 0)
        def _():
            pltpu.async_copy(rel_hbm, rel_sp, rsem).wait()

        c1 = pltpu.async_copy(qe_hbm.at[pl.ds(row0, NCHUNK)], qe_v, isem)
        c2 = pltpu.async_copy(qr_hbm.at[pl.ds(row0, NCHUNK)], qr_v, isem)
        c3 = pltpu.async_copy(oe_hbm.at[pl.ds(row0, NCHUNK)], oe_v, isem)
        c1.wait(); c2.wait(); c3.wait()

        tasks = []
        for iv, tab, out in ((qe_v, ent_hbm, out_qe),
                             (oe_v, ent_hbm, out_oe),
                             (qr_v, rel_sp, out_qr)):
            for j in range(NCHUNK):
                tasks.append((iv.at[j], tab, out, (row0 + j) * CHUNK))

        gcp = [None] * T
        scp = [None] * T

        def fire(t):
            if t == TE:
                plsc.subcore_barrier()
            iv_row, tab, _, _ = tasks[t]
            gcp[t] = pltpu.async_copy(tab.at[iv_row], bufs[t % NBUF],
                                      gsem[t % NBUF])

        for t in range(NBUF):
            fire(t)
        for t in range(T):
            b = t % NBUF
            _, _, out, off = tasks[t]
            gcp[t].wait()
            scp[t] = pltpu.async_copy(bufs[b], out.at[pl.ds(off, CHUNK)],
                                      ssem[b])
            if t + NBUF < T:
                scp[t].wait()
                fire(t + NBUF)
        for t in range(T - NBUF, T):
            scp[t].wait()

    return k(qe, qr, oe, ent_table, rel_table)


def kernel(query_entities, query_relations, obj_entities, ent_table, rel_table):
    qe = query_entities.astype(jnp.int32).reshape(B // CHUNK, CHUNK)
    qr = query_relations.astype(jnp.int32).reshape(B // CHUNK, CHUNK)
    oe = obj_entities.astype(jnp.int32).reshape(B // CHUNK, CHUNK)
    out_qe, out_qr, out_oe = _gather3(qe, qr, oe, ent_table, rel_table)
    return (out_qe, out_qr, out_oe)
